# baseline (speedup 1.0000x reference)
import functools

import jax
import jax.numpy as jnp
from jax import lax
from jax.experimental import pallas as pl
from jax.experimental.pallas import tpu as pltpu
from jax.experimental.pallas import tpu_sc as plsc

D_MODEL = 64
SCALE = 8.0

NC = 2
NS = 16
NW = NC * NS
LANES = 16

CHUNK = 128
NB = 4


def _sc_embed(xt2d, table, n_chunks, seq):
    per_w = n_chunks // NW
    assert per_w % NB == 0
    iblocks = n_chunks // seq
    mesh = plsc.VectorSubcoreMesh(core_axis_name="c", subcore_axis_name="s")

    scratch = (
        [pltpu.VMEM((per_w, CHUNK), jnp.int32)]
        + [pltpu.VMEM((CHUNK, D_MODEL), jnp.float32) for _ in range(NB)]
        + [pltpu.VMEM((D_MODEL // 8, 1024), jnp.float32) for _ in range(NB)]
        + [pltpu.SemaphoreType.DMA for _ in range(2 * NB)]
    )

    @functools.partial(
        pl.kernel,
        out_type=jax.ShapeDtypeStruct((seq, D_MODEL // 8, CHUNK, 1024),
                                      jnp.float32),
        mesh=mesh,
        scratch_types=scratch,
        compiler_params=pltpu.CompilerParams(
            use_tc_tiling_on_sc=False, needs_layout_passes=False),
    )
    def k(x_hbm, table_hbm, out_hbm, idx_v, *bufs_and_sems):
        inb = bufs_and_sems[:NB]
        trb = bufs_and_sems[NB:2 * NB]
        sem_in = bufs_and_sems[2 * NB:3 * NB]
        sem_out = bufs_and_sems[3 * NB:4 * NB]

        wid = lax.axis_index("s") * NC + lax.axis_index("c")
        base = wid * per_w
        pltpu.sync_copy(x_hbm.at[pl.ds(base, per_w)], idx_v)

        row_iota = lax.iota(jnp.int32, LANES)

        for b in range(NB):
            pltpu.async_copy(table_hbm.at[idx_v.at[b]], inb[b], sem_in[b])

        @pl.loop(0, per_w, step=NB)
        def _outer(i0):
            for b in range(NB):
                c = i0 + b
                g = base + c
                j = g // CHUNK
                iblk = g % CHUNK

                pltpu.make_async_copy(
                    table_hbm.at[idx_v.at[c]], inb[b], sem_in[b]).wait()

                @pl.when(c >= NB)
                def _():
                    pltpu.make_async_copy(
                        trb[b], out_hbm.at[0, pl.ds(0, D_MODEL // 8), 0],
                        sem_out[b]).wait()

                @pl.loop(0, D_MODEL, unroll=8)
                def _col(kk):
                    kb = kk // 8
                    k8 = kk % 8
                    col = jnp.broadcast_to(kb * 8 + k8, (LANES,))
                    for v in range(CHUNK // LANES):
                        rows = row_iota + (v * LANES)
                        vals = plsc.load_gather(inb[b], [rows, col])
                        trb[b][kb, pl.ds(k8 * 128 + v * LANES, LANES)] = (
                            vals * SCALE)

                @pl.when(c + NB < per_w)
                def _():
                    pltpu.async_copy(
                        table_hbm.at[idx_v.at[c + NB]], inb[b], sem_in[b])

                pltpu.async_copy(
                    trb[b], out_hbm.at[j, pl.ds(0, D_MODEL // 8), iblk],
                    sem_out[b])

        for b in range(NB):
            pltpu.make_async_copy(
                trb[b], out_hbm.at[0, pl.ds(0, D_MODEL // 8), 0],
                sem_out[b]).wait()

    return k(xt2d, table)


def kernel(x, table):
    bsz, seq = x.shape
    iblocks = bsz // CHUNK
    n_chunks = seq * iblocks
    xt2d = x.T.reshape(n_chunks, CHUNK).astype(jnp.int32)
    out4d = _sc_embed(xt2d, table, n_chunks, seq)
    o5 = out4d.reshape(seq, 8, iblocks, 8, CHUNK)
    return o5.transpose(2, 4, 0, 1, 3).reshape(bsz, seq, D_MODEL)


# speedup vs baseline: 1.1385x; 1.1385x over previous
import functools

import jax
import jax.numpy as jnp
from jax import lax
from jax.experimental import pallas as pl
from jax.experimental.pallas import tpu as pltpu
from jax.experimental.pallas import tpu_sc as plsc

D_MODEL = 64
SCALE = 8.0

NC = 2
NS = 16
NW = NC * NS
LANES = 16

CHUNK = 128
NB = 4


def _sc_embed(xt2d, table, n_chunks, seq):
    per_w = n_chunks // NW
    assert per_w % NB == 0
    iblocks = n_chunks // seq
    mesh = plsc.VectorSubcoreMesh(core_axis_name="c", subcore_axis_name="s")

    scratch = (
        [pltpu.VMEM((per_w, CHUNK), jnp.int32)]
        + [pltpu.VMEM((CHUNK, D_MODEL), jnp.float32) for _ in range(NB)]
        + [pltpu.VMEM((D_MODEL // 8, 1024), jnp.float32) for _ in range(NB)]
        + [pltpu.SemaphoreType.DMA for _ in range(2 * NB)]
    )

    @functools.partial(
        pl.kernel,
        out_type=jax.ShapeDtypeStruct((seq, D_MODEL // 8, CHUNK, 1024),
                                      jnp.float32),
        mesh=mesh,
        scratch_types=scratch,
        compiler_params=pltpu.CompilerParams(
            use_tc_tiling_on_sc=False, needs_layout_passes=False),
    )
    def k(x_hbm, table_hbm, out_hbm, idx_v, *bufs_and_sems):
        inb = bufs_and_sems[:NB]
        trb = bufs_and_sems[NB:2 * NB]
        sem_in = bufs_and_sems[2 * NB:3 * NB]
        sem_out = bufs_and_sems[3 * NB:4 * NB]

        wid = lax.axis_index("s") * NC + lax.axis_index("c")
        base = wid * per_w
        pltpu.sync_copy(x_hbm.at[pl.ds(base, per_w)], idx_v)

        row_iota = lax.iota(jnp.int32, LANES)
        p_row = [(row_iota + w * LANES) // 8 for w in range(D_MODEL // LANES)]
        p_col = [((row_iota + w * LANES) % 8) * 128
                 for w in range(D_MODEL // LANES)]

        for b in range(NB):
            pltpu.async_copy(table_hbm.at[idx_v.at[b]], inb[b], sem_in[b])

        @pl.loop(0, per_w, step=NB)
        def _outer(i0):
            for b in range(NB):
                c = i0 + b
                g = base + c
                j = g // CHUNK
                iblk = g % CHUNK

                pltpu.make_async_copy(
                    table_hbm.at[idx_v.at[c]], inb[b], sem_in[b]).wait()

                @pl.when(c >= NB)
                def _():
                    pltpu.make_async_copy(
                        trb[b], out_hbm.at[0, pl.ds(0, D_MODEL // 8), 0],
                        sem_out[b]).wait()

                @pl.loop(0, CHUNK, unroll=4)
                def _row(ii):
                    col_off = jnp.broadcast_to(ii, (LANES,))
                    for w in range(D_MODEL // LANES):
                        vals = inb[b][ii, pl.ds(w * LANES, LANES)] * SCALE
                        plsc.store_scatter(
                            trb[b], [p_row[w], p_col[w] + col_off], vals)

                @pl.when(c + NB < per_w)
                def _():
                    pltpu.async_copy(
                        table_hbm.at[idx_v.at[c + NB]], inb[b], sem_in[b])

                pltpu.async_copy(
                    trb[b], out_hbm.at[j, pl.ds(0, D_MODEL // 8), iblk],
                    sem_out[b])

        for b in range(NB):
            pltpu.make_async_copy(
                trb[b], out_hbm.at[0, pl.ds(0, D_MODEL // 8), 0],
                sem_out[b]).wait()

    return k(xt2d, table)


def kernel(x, table):
    bsz, seq = x.shape
    iblocks = bsz // CHUNK
    n_chunks = seq * iblocks
    xt2d = x.T.reshape(n_chunks, CHUNK).astype(jnp.int32)
    out4d = _sc_embed(xt2d, table, n_chunks, seq)
    o5 = out4d.reshape(seq, 8, iblocks, 8, CHUNK)
    return o5.transpose(2, 4, 0, 1, 3).reshape(bsz, seq, D_MODEL)


# speedup vs baseline: 1.2762x; 1.1210x over previous
import functools

import jax
import jax.numpy as jnp
from jax import lax
from jax.experimental import pallas as pl
from jax.experimental.pallas import tpu as pltpu
from jax.experimental.pallas import tpu_sc as plsc

D_MODEL = 64
SCALE = 8.0

NC = 2
NS = 16
NW = NC * NS

CHUNK = 128
NB = 4
PF = NB - 1


def _sc_gather(x2d, table, n_chunks):
    per_w = n_chunks // NW
    mesh = plsc.VectorSubcoreMesh(core_axis_name="c", subcore_axis_name="s")

    scratch = (
        [pltpu.VMEM((per_w, CHUNK), jnp.int32)]
        + [pltpu.VMEM((CHUNK, D_MODEL), jnp.float32) for _ in range(NB)]
        + [pltpu.SemaphoreType.DMA for _ in range(2 * NB)]
    )

    @functools.partial(
        pl.kernel,
        out_type=jax.ShapeDtypeStruct((n_chunks * CHUNK, D_MODEL),
                                      jnp.float32),
        mesh=mesh,
        scratch_types=scratch,
        compiler_params=pltpu.CompilerParams(
            use_tc_tiling_on_sc=False, needs_layout_passes=False),
    )
    def k(x_hbm, table_hbm, out_hbm, idx_v, *bufs_and_sems):
        inb = bufs_and_sems[:NB]
        sem_in = bufs_and_sems[NB:2 * NB]
        sem_out = bufs_and_sems[2 * NB:3 * NB]

        wid = lax.axis_index("s") * NC + lax.axis_index("c")
        base = wid * per_w
        pltpu.sync_copy(x_hbm.at[pl.ds(base, per_w)], idx_v)

        def gather(c, b):
            pltpu.async_copy(table_hbm.at[idx_v.at[c]], inb[b], sem_in[b])

        for c in range(PF):
            gather(c, c % NB)

        @pl.loop(0, per_w, step=NB)
        def _outer(i):
            for b in range(NB):
                c = i + b
                bp = (b + PF) % NB

                @pl.when(jnp.logical_and(c + PF >= NB, c + PF < per_w))
                def _():
                    pltpu.make_async_copy(
                        inb[bp], out_hbm.at[pl.ds(0, CHUNK)],
                        sem_out[bp]).wait()

                @pl.when(c + PF < per_w)
                def _():
                    gather(c + PF, bp)

                pltpu.make_async_copy(
                    table_hbm.at[idx_v.at[c]], inb[b], sem_in[b]).wait()
                pltpu.async_copy(
                    inb[b], out_hbm.at[pl.ds((base + c) * CHUNK, CHUNK)],
                    sem_out[b])

        for b in range(min(NB, per_w)):
            pltpu.make_async_copy(
                inb[b], out_hbm.at[pl.ds(0, CHUNK)], sem_out[b]).wait()

    return k(x2d, table)


def _tc_transform(lin2, seq, iblocks):
    grp = 8

    def body(in_ref, out_ref):
        for c in range(grp):
            blk = in_ref[c * 64:(c + 1) * 64, :]
            a = blk[:, 0:D_MODEL]
            b = blk[:, D_MODEL:2 * D_MODEL]
            out_ref[0, :, c * CHUNK:c * CHUNK + 64] = a.T * SCALE
            out_ref[0, :, c * CHUNK + 64:(c + 1) * CHUNK] = b.T * SCALE

    return pl.pallas_call(
        body,
        out_shape=jax.ShapeDtypeStruct((seq, D_MODEL, iblocks * CHUNK),
                                       jnp.float32),
        grid=(seq, iblocks // grp),
        in_specs=[pl.BlockSpec((grp * 64, 128),
                               lambda j, g: (j * (iblocks // grp) + g, 0))],
        out_specs=pl.BlockSpec((1, D_MODEL, grp * CHUNK),
                               lambda j, g: (j, 0, g)),
    )(lin2)


def kernel(x, table):
    bsz, seq = x.shape
    iblocks = bsz // CHUNK
    n_chunks = seq * iblocks
    xt = x.T.reshape(n_chunks, 2, 64).transpose(0, 2, 1).reshape(
        n_chunks, CHUNK).astype(jnp.int32)
    lin = _sc_gather(xt, table, n_chunks)
    lin2 = lin.reshape(n_chunks * D_MODEL, 128)
    out3 = _tc_transform(lin2, seq, iblocks)
    return out3.transpose(2, 0, 1)


# speedup vs baseline: 1.2776x; 1.0010x over previous
import functools

import jax
import jax.numpy as jnp
from jax import lax
from jax.experimental import pallas as pl
from jax.experimental.pallas import tpu as pltpu
from jax.experimental.pallas import tpu_sc as plsc

D_MODEL = 64
SCALE = 8.0

NC = 2
NS = 16
NW = NC * NS

CHUNK = 128
NB = 4
PF = NB - 1


def _sc_gather(x2d, table, n_chunks):
    per_w = n_chunks // NW
    mesh = plsc.VectorSubcoreMesh(core_axis_name="c", subcore_axis_name="s")

    scratch = (
        [pltpu.VMEM((per_w, CHUNK), jnp.int32)]
        + [pltpu.VMEM((CHUNK, D_MODEL), jnp.float32) for _ in range(NB)]
        + [pltpu.SemaphoreType.DMA for _ in range(2 * NB)]
    )

    @functools.partial(
        pl.kernel,
        out_type=jax.ShapeDtypeStruct((n_chunks * CHUNK, D_MODEL),
                                      jnp.float32),
        mesh=mesh,
        scratch_types=scratch,
        compiler_params=pltpu.CompilerParams(
            use_tc_tiling_on_sc=False, needs_layout_passes=False),
    )
    def k(x_hbm, table_hbm, out_hbm, idx_v, *bufs_and_sems):
        inb = bufs_and_sems[:NB]
        sem_in = bufs_and_sems[NB:2 * NB]
        sem_out = bufs_and_sems[2 * NB:3 * NB]

        wid = lax.axis_index("s") * NC + lax.axis_index("c")
        base = wid * per_w
        pltpu.sync_copy(x_hbm.at[pl.ds(base, per_w)], idx_v)

        def gather(c, b):
            pltpu.async_copy(table_hbm.at[idx_v.at[c]], inb[b], sem_in[b])

        for c in range(NB):
            gather(c, c)

        @pl.loop(0, per_w, step=NB)
        def _outer(i):
            for b in range(NB):
                c = i + b

                pltpu.make_async_copy(
                    table_hbm.at[idx_v.at[c]], inb[b], sem_in[b]).wait()

                pltpu.async_copy(
                    inb[b], out_hbm.at[pl.ds((base + c) * CHUNK, CHUNK)],
                    sem_out[b])

                @pl.when(c + NB < per_w)
                def _():
                    pltpu.make_async_copy(
                        inb[b], out_hbm.at[pl.ds(0, CHUNK)],
                        sem_out[b]).wait()
                    gather(c + NB, b)

        for b in range(min(NB, per_w)):
            pltpu.make_async_copy(
                inb[b], out_hbm.at[pl.ds(0, CHUNK)], sem_out[b]).wait()

    return k(x2d, table)


def _tc_transform(lin2, seq, iblocks):
    grp = 8

    def body(in_ref, out_ref):
        for c in range(grp):
            blk = in_ref[c * 64:(c + 1) * 64, :]
            a = blk[:, 0:D_MODEL]
            b = blk[:, D_MODEL:2 * D_MODEL]
            out_ref[0, :, c * CHUNK:c * CHUNK + 64] = a.T * SCALE
            out_ref[0, :, c * CHUNK + 64:(c + 1) * CHUNK] = b.T * SCALE

    return pl.pallas_call(
        body,
        out_shape=jax.ShapeDtypeStruct((seq, D_MODEL, iblocks * CHUNK),
                                       jnp.float32),
        grid=(seq, iblocks // grp),
        in_specs=[pl.BlockSpec((grp * 64, 128),
                               lambda j, g: (j * (iblocks // grp) + g, 0))],
        out_specs=pl.BlockSpec((1, D_MODEL, grp * CHUNK),
                               lambda j, g: (j, 0, g)),
    )(lin2)


def kernel(x, table):
    bsz, seq = x.shape
    iblocks = bsz // CHUNK
    n_chunks = seq * iblocks
    xt = x.T.reshape(n_chunks, 2, 64).transpose(0, 2, 1).reshape(
        n_chunks, CHUNK).astype(jnp.int32)
    lin = _sc_gather(xt, table, n_chunks)
    lin2 = lin.reshape(n_chunks * D_MODEL, 128)
    out3 = _tc_transform(lin2, seq, iblocks)
    return out3.transpose(2, 0, 1)


# speedup vs baseline: 1.4751x; 1.1546x over previous
import functools

import jax
import jax.numpy as jnp
from jax import lax
from jax.experimental import pallas as pl
from jax.experimental.pallas import tpu as pltpu
from jax.experimental.pallas import tpu_sc as plsc

D_MODEL = 64
SCALE = 8.0

NC = 2
NS = 16
NW = NC * NS

CHUNK = 128
NB = 4
PF = NB - 1


def _sc_gather(x2d, table, n_chunks):
    per_w = n_chunks // NW
    mesh = plsc.VectorSubcoreMesh(core_axis_name="c", subcore_axis_name="s")

    scratch = (
        [pltpu.VMEM((per_w, CHUNK), jnp.int32)]
        + [pltpu.VMEM((CHUNK, D_MODEL), jnp.float32) for _ in range(NB)]
        + [pltpu.SemaphoreType.DMA for _ in range(2 * NB)]
    )

    @functools.partial(
        pl.kernel,
        out_type=jax.ShapeDtypeStruct((n_chunks * CHUNK, D_MODEL),
                                      jnp.float32),
        mesh=mesh,
        scratch_types=scratch,
        compiler_params=pltpu.CompilerParams(
            use_tc_tiling_on_sc=False, needs_layout_passes=False),
    )
    def k(x_hbm, table_hbm, out_hbm, idx_v, *bufs_and_sems):
        inb = bufs_and_sems[:NB]
        sem_in = bufs_and_sems[NB:2 * NB]
        sem_out = bufs_and_sems[2 * NB:3 * NB]

        wid = lax.axis_index("s") * NC + lax.axis_index("c")
        base = wid * per_w
        pltpu.sync_copy(x_hbm.at[pl.ds(base, per_w)], idx_v)

        def gather(c, b):
            pltpu.async_copy(table_hbm.at[idx_v.at[c]], inb[b], sem_in[b])

        for c in range(NB):
            gather(c, c)

        @pl.loop(0, per_w, step=NB)
        def _outer(i):
            for b in range(NB):
                c = i + b

                pltpu.make_async_copy(
                    table_hbm.at[idx_v.at[c]], inb[b], sem_in[b]).wait()

                pltpu.async_copy(
                    inb[b], out_hbm.at[pl.ds((base + c) * CHUNK, CHUNK)],
                    sem_out[b])

                @pl.when(c + NB < per_w)
                def _():
                    pltpu.make_async_copy(
                        inb[b], out_hbm.at[pl.ds(0, CHUNK)],
                        sem_out[b]).wait()
                    gather(c + NB, b)

        for b in range(min(NB, per_w)):
            pltpu.make_async_copy(
                inb[b], out_hbm.at[pl.ds(0, CHUNK)], sem_out[b]).wait()

    return k(x2d, table)


def _tc_transform(lin2, seq, iblocks):
    grp = 8

    def body(in_ref, out_ref):
        for c in range(grp):
            blk = in_ref[c * 64:(c + 1) * 64, :]
            t = blk.T * SCALE
            out_ref[0, :, c * CHUNK:(c + 1) * CHUNK] = jnp.concatenate(
                [t[0:D_MODEL, :], t[D_MODEL:2 * D_MODEL, :]], axis=1)

    return pl.pallas_call(
        body,
        out_shape=jax.ShapeDtypeStruct((seq, D_MODEL, iblocks * CHUNK),
                                       jnp.float32),
        grid=(seq, iblocks // grp),
        in_specs=[pl.BlockSpec((grp * 64, 128),
                               lambda j, g: (j * (iblocks // grp) + g, 0))],
        out_specs=pl.BlockSpec((1, D_MODEL, grp * CHUNK),
                               lambda j, g: (j, 0, g)),
        compiler_params=pltpu.CompilerParams(
            dimension_semantics=("parallel", "parallel")),
    )(lin2)


def kernel(x, table):
    bsz, seq = x.shape
    iblocks = bsz // CHUNK
    n_chunks = seq * iblocks
    xt = x.T.reshape(n_chunks, 2, 64).transpose(0, 2, 1).reshape(
        n_chunks, CHUNK).astype(jnp.int32)
    lin = _sc_gather(xt, table, n_chunks)
    lin2 = lin.reshape(n_chunks * D_MODEL, 128)
    out3 = _tc_transform(lin2, seq, iblocks)
    return out3.transpose(2, 0, 1)


# speedup vs baseline: 1.7892x; 1.2129x over previous
import functools

import jax
import jax.numpy as jnp
from jax import lax
from jax.experimental import pallas as pl
from jax.experimental.pallas import tpu as pltpu
from jax.experimental.pallas import tpu_sc as plsc

D_MODEL = 64
SCALE = 8.0

NC = 2
NS = 16
NW = NC * NS

CHUNK = 128
NB = 4
PF = NB - 1


def _sc_gather(x2d, table, n_chunks):
    per_w = n_chunks // NW
    mesh = plsc.VectorSubcoreMesh(core_axis_name="c", subcore_axis_name="s")

    scratch = (
        [pltpu.VMEM((per_w, CHUNK), jnp.int32)]
        + [pltpu.VMEM((CHUNK, D_MODEL), jnp.float32) for _ in range(NB)]
        + [pltpu.SemaphoreType.DMA for _ in range(2 * NB)]
    )

    @functools.partial(
        pl.kernel,
        out_type=jax.ShapeDtypeStruct((n_chunks * CHUNK, D_MODEL),
                                      jnp.float32),
        mesh=mesh,
        scratch_types=scratch,
        compiler_params=pltpu.CompilerParams(
            use_tc_tiling_on_sc=False, needs_layout_passes=False),
    )
    def k(x_hbm, table_hbm, out_hbm, idx_v, *bufs_and_sems):
        inb = bufs_and_sems[:NB]
        sem_in = bufs_and_sems[NB:2 * NB]
        sem_out = bufs_and_sems[2 * NB:3 * NB]

        wid = lax.axis_index("s") * NC + lax.axis_index("c")
        base = wid * per_w
        pltpu.sync_copy(x_hbm.at[pl.ds(base, per_w)], idx_v)

        def gather(c, b):
            pltpu.async_copy(table_hbm.at[idx_v.at[c]], inb[b], sem_in[b])

        for c in range(NB):
            gather(c, c)

        @pl.loop(0, per_w, step=NB)
        def _outer(i):
            for b in range(NB):
                c = i + b

                pltpu.make_async_copy(
                    table_hbm.at[idx_v.at[c]], inb[b], sem_in[b]).wait()

                pltpu.async_copy(
                    inb[b], out_hbm.at[pl.ds((base + c) * CHUNK, CHUNK)],
                    sem_out[b])

                @pl.when(c + NB < per_w)
                def _():
                    pltpu.make_async_copy(
                        inb[b], out_hbm.at[pl.ds(0, CHUNK)],
                        sem_out[b]).wait()
                    gather(c + NB, b)

        for b in range(min(NB, per_w)):
            pltpu.make_async_copy(
                inb[b], out_hbm.at[pl.ds(0, CHUNK)], sem_out[b]).wait()

    return k(x2d, table)


def _tc_transform(lin2, seq, iblocks):
    grp = 16

    def body(in_ref, out_ref):
        bt = in_ref[...].T * SCALE
        for c in range(grp):
            out_ref[0, :, c * CHUNK:(c + 1) * CHUNK] = jnp.concatenate(
                [bt[0:D_MODEL, c * 64:(c + 1) * 64],
                 bt[D_MODEL:2 * D_MODEL, c * 64:(c + 1) * 64]], axis=1)

    return pl.pallas_call(
        body,
        out_shape=jax.ShapeDtypeStruct((seq, D_MODEL, iblocks * CHUNK),
                                       jnp.float32),
        grid=(seq, iblocks // grp),
        in_specs=[pl.BlockSpec((grp * 64, 128),
                               lambda j, g: (j * (iblocks // grp) + g, 0))],
        out_specs=pl.BlockSpec((1, D_MODEL, grp * CHUNK),
                               lambda j, g: (j, 0, g)),
        compiler_params=pltpu.CompilerParams(
            dimension_semantics=("parallel", "parallel")),
    )(lin2)


def kernel(x, table):
    bsz, seq = x.shape
    iblocks = bsz // CHUNK
    n_chunks = seq * iblocks
    xt = x.T.reshape(n_chunks, 2, 64).transpose(0, 2, 1).reshape(
        n_chunks, CHUNK).astype(jnp.int32)
    lin = _sc_gather(xt, table, n_chunks)
    lin2 = lin.reshape(n_chunks * D_MODEL, 128)
    out3 = _tc_transform(lin2, seq, iblocks)
    return out3.transpose(2, 0, 1)


# speedup vs baseline: 1.9564x; 1.0935x over previous
import functools

import jax
import jax.numpy as jnp
from jax import lax
from jax.experimental import pallas as pl
from jax.experimental.pallas import tpu as pltpu
from jax.experimental.pallas import tpu_sc as plsc

D_MODEL = 64
SCALE = 8.0

NC = 2
NS = 16
NW = NC * NS

CHUNK = 128
NB = 4
PF = NB - 1


def _sc_gather(x2d, table, n_chunks):
    per_w = n_chunks // NW
    mesh = plsc.VectorSubcoreMesh(core_axis_name="c", subcore_axis_name="s")

    scratch = (
        [pltpu.VMEM((per_w, CHUNK), jnp.int32)]
        + [pltpu.VMEM((CHUNK, D_MODEL), jnp.float32) for _ in range(NB)]
        + [pltpu.SemaphoreType.DMA for _ in range(2 * NB)]
    )

    @functools.partial(
        pl.kernel,
        out_type=jax.ShapeDtypeStruct((n_chunks * CHUNK, D_MODEL),
                                      jnp.float32),
        mesh=mesh,
        scratch_types=scratch,
        compiler_params=pltpu.CompilerParams(
            use_tc_tiling_on_sc=False, needs_layout_passes=False),
    )
    def k(x_hbm, table_hbm, out_hbm, idx_v, *bufs_and_sems):
        inb = bufs_and_sems[:NB]
        sem_in = bufs_and_sems[NB:2 * NB]
        sem_out = bufs_and_sems[2 * NB:3 * NB]

        wid = lax.axis_index("s") * NC + lax.axis_index("c")
        base = wid * per_w
        pltpu.sync_copy(x_hbm.at[pl.ds(base, per_w)], idx_v)

        def gather(c, b):
            pltpu.async_copy(table_hbm.at[idx_v.at[c]], inb[b], sem_in[b])

        for c in range(NB):
            gather(c, c)

        @pl.loop(0, per_w, step=NB)
        def _outer(i):
            for b in range(NB):
                c = i + b

                pltpu.make_async_copy(
                    table_hbm.at[idx_v.at[c]], inb[b], sem_in[b]).wait()

                pltpu.async_copy(
                    inb[b], out_hbm.at[pl.ds((base + c) * CHUNK, CHUNK)],
                    sem_out[b])

                @pl.when(c + NB < per_w)
                def _():
                    pltpu.make_async_copy(
                        inb[b], out_hbm.at[pl.ds(0, CHUNK)],
                        sem_out[b]).wait()
                    gather(c + NB, b)

        for b in range(min(NB, per_w)):
            pltpu.make_async_copy(
                inb[b], out_hbm.at[pl.ds(0, CHUNK)], sem_out[b]).wait()

    return k(x2d, table)


def _tc_transform(lin2, seq, iblocks):
    grp = 32

    def body(in_ref, out_ref):
        bt = in_ref[...].T * SCALE
        for c in range(grp):
            out_ref[0, :, c * CHUNK:(c + 1) * CHUNK] = jnp.concatenate(
                [bt[0:D_MODEL, c * 64:(c + 1) * 64],
                 bt[D_MODEL:2 * D_MODEL, c * 64:(c + 1) * 64]], axis=1)

    return pl.pallas_call(
        body,
        out_shape=jax.ShapeDtypeStruct((seq, D_MODEL, iblocks * CHUNK),
                                       jnp.float32),
        grid=(seq, iblocks // grp),
        in_specs=[pl.BlockSpec((grp * 64, 128),
                               lambda j, g: (j * (iblocks // grp) + g, 0))],
        out_specs=pl.BlockSpec((1, D_MODEL, grp * CHUNK),
                               lambda j, g: (j, 0, g)),
        compiler_params=pltpu.CompilerParams(
            dimension_semantics=("parallel", "parallel")),
    )(lin2)


def kernel(x, table):
    bsz, seq = x.shape
    iblocks = bsz // CHUNK
    n_chunks = seq * iblocks
    xt = x.T.reshape(n_chunks, 2, 64).transpose(0, 2, 1).reshape(
        n_chunks, CHUNK).astype(jnp.int32)
    lin = _sc_gather(xt, table, n_chunks)
    lin2 = lin.reshape(n_chunks * D_MODEL, 128)
    out3 = _tc_transform(lin2, seq, iblocks)
    return out3.transpose(2, 0, 1)


# speedup vs baseline: 2.0785x; 1.0624x over previous
import functools

import jax
import jax.numpy as jnp
from jax import lax
from jax.experimental import pallas as pl
from jax.experimental.pallas import tpu as pltpu
from jax.experimental.pallas import tpu_sc as plsc

D_MODEL = 64
SCALE = 8.0

NC = 2
NS = 16
NW = NC * NS

CHUNK = 128
NB = 4
PF = NB - 1


def _sc_gather(x2d, table, n_chunks):
    per_w = n_chunks // NW
    mesh = plsc.VectorSubcoreMesh(core_axis_name="c", subcore_axis_name="s")

    scratch = (
        [pltpu.VMEM((per_w, CHUNK), jnp.int32)]
        + [pltpu.VMEM((CHUNK, D_MODEL), jnp.float32) for _ in range(NB)]
        + [pltpu.SemaphoreType.DMA for _ in range(2 * NB)]
    )

    @functools.partial(
        pl.kernel,
        out_type=jax.ShapeDtypeStruct((n_chunks * CHUNK, D_MODEL),
                                      jnp.float32),
        mesh=mesh,
        scratch_types=scratch,
        compiler_params=pltpu.CompilerParams(
            use_tc_tiling_on_sc=False, needs_layout_passes=False),
    )
    def k(x_hbm, table_hbm, out_hbm, idx_v, *bufs_and_sems):
        inb = bufs_and_sems[:NB]
        sem_in = bufs_and_sems[NB:2 * NB]
        sem_out = bufs_and_sems[2 * NB:3 * NB]

        wid = lax.axis_index("s") * NC + lax.axis_index("c")
        base = wid * per_w
        pltpu.sync_copy(x_hbm.at[pl.ds(base, per_w)], idx_v)

        def gather(c, b):
            pltpu.async_copy(table_hbm.at[idx_v.at[c]], inb[b], sem_in[b])

        for c in range(NB):
            gather(c, c)

        @pl.loop(0, per_w, step=NB)
        def _outer(i):
            for b in range(NB):
                c = i + b

                pltpu.make_async_copy(
                    table_hbm.at[idx_v.at[c]], inb[b], sem_in[b]).wait()

                pltpu.async_copy(
                    inb[b], out_hbm.at[pl.ds((base + c) * CHUNK, CHUNK)],
                    sem_out[b])

                @pl.when(c + NB < per_w)
                def _():
                    pltpu.make_async_copy(
                        inb[b], out_hbm.at[pl.ds(0, CHUNK)],
                        sem_out[b]).wait()
                    gather(c + NB, b)

        for b in range(min(NB, per_w)):
            pltpu.make_async_copy(
                inb[b], out_hbm.at[pl.ds(0, CHUNK)], sem_out[b]).wait()

    return k(x2d, table)


def _tc_transform(lin2, seq, iblocks):
    grp = 64

    def body(in_ref, out_ref):
        bt = in_ref[...].T * SCALE
        for c in range(grp):
            out_ref[0, :, c * CHUNK:(c + 1) * CHUNK] = jnp.concatenate(
                [bt[0:D_MODEL, c * 64:(c + 1) * 64],
                 bt[D_MODEL:2 * D_MODEL, c * 64:(c + 1) * 64]], axis=1)

    return pl.pallas_call(
        body,
        out_shape=jax.ShapeDtypeStruct((seq, D_MODEL, iblocks * CHUNK),
                                       jnp.float32),
        grid=(seq, iblocks // grp),
        in_specs=[pl.BlockSpec((grp * 64, 128),
                               lambda j, g: (j * (iblocks // grp) + g, 0))],
        out_specs=pl.BlockSpec((1, D_MODEL, grp * CHUNK),
                               lambda j, g: (j, 0, g)),
        compiler_params=pltpu.CompilerParams(
            dimension_semantics=("parallel", "parallel")),
    )(lin2)


def kernel(x, table):
    bsz, seq = x.shape
    iblocks = bsz // CHUNK
    n_chunks = seq * iblocks
    xt = x.T.reshape(n_chunks, 2, 64).transpose(0, 2, 1).reshape(
        n_chunks, CHUNK).astype(jnp.int32)
    lin = _sc_gather(xt, table, n_chunks)
    lin2 = lin.reshape(n_chunks * D_MODEL, 128)
    out3 = _tc_transform(lin2, seq, iblocks)
    return out3.transpose(2, 0, 1)
